# baseline (speedup 1.0000x reference)
import functools

import jax
import jax.numpy as jnp
from jax import lax
from jax.experimental import pallas as pl
from jax.experimental.pallas import tpu as pltpu
from jax.experimental.pallas import tpu_sc as plsc

EMB_DIM = 64
BATCH = 16384
NEG = 20
NTILES = 32
ITEMS_PER_TILE = BATCH // NTILES
SUB = 32
NSUB = ITEMS_PER_TILE // SUB
NEG_ROWS = SUB * NEG
PART = 128
NEG_PARTS = NEG_ROWS // PART
OUT_COLS = 48
NCOLS = 2 + 2 * NEG
LG = 16
DCHUNK = 16


def _sc_scores(u_l, u_r, v_l, v_r, pu2, pvl2, pvr2, nl2, nr2):
    mesh = plsc.VectorSubcoreMesh(core_axis_name="c", subcore_axis_name="s")

    @functools.partial(
        pl.kernel,
        out_type=jax.ShapeDtypeStruct((BATCH // SUB, OUT_COLS, SUB), jnp.float32),
        mesh=mesh,
        compiler_params=pltpu.CompilerParams(
            needs_layout_passes=False, use_tc_tiling_on_sc=False),
        scratch_types=[
            pltpu.VMEM((NSUB, SUB), jnp.int32),
            pltpu.VMEM((NSUB, SUB), jnp.int32),
            pltpu.VMEM((NSUB, SUB), jnp.int32),
            pltpu.VMEM((NSUB * NEG_PARTS, PART), jnp.int32),
            pltpu.VMEM((NSUB * NEG_PARTS, PART), jnp.int32),
            pltpu.VMEM((ITEMS_PER_TILE,), jnp.int32),
            pltpu.VMEM((NEG, ITEMS_PER_TILE // 2), jnp.int32),
            pltpu.VMEM((SUB, EMB_DIM), jnp.float32),
            pltpu.VMEM((SUB, EMB_DIM), jnp.float32),
            pltpu.VMEM((NEG_ROWS, EMB_DIM), jnp.float32),
            pltpu.VMEM((SUB, EMB_DIM), jnp.float32),
            pltpu.VMEM((SUB, EMB_DIM), jnp.float32),
            pltpu.VMEM((NEG_ROWS, EMB_DIM), jnp.float32),
            pltpu.VMEM((OUT_COLS, SUB), jnp.float32),
            pltpu.VMEM((OUT_COLS, SUB), jnp.float32),
            pltpu.SemaphoreType.DMA,
            pltpu.SemaphoreType.DMA,
            pltpu.SemaphoreType.DMA,
            pltpu.SemaphoreType.DMA,
        ],
    )
    def k(u_l_h, u_r_h, v_l_h, v_r_h, pu_h, pvl_h, pvr_h, nl_h, nr_h, out_h,
          pu_v, pvl_v, pvr_v, nl_v, nr_v, praw, nraw,
          eul, evl, enl, eur, evr, enr,
          scoA, scoB, semL, semR, semOA, semOB):
        wid = lax.axis_index("s") * 2 + lax.axis_index("c")
        base_item = wid * ITEMS_PER_TILE

        for src_h, dst in ((pu_h, pu_v), (pvl_h, pvl_v), (pvr_h, pvr_v)):
            pltpu.sync_copy(src_h.at[pl.ds(base_item, ITEMS_PER_TILE)], praw)
            for j in range(NSUB):
                for h in range(SUB // LG):
                    dst[j, pl.ds(h * LG, LG)] = praw[pl.ds(j * SUB + h * LG, LG)]
        HALF_ITEMS = ITEMS_PER_TILE // 2
        HALF_SUBS = NSUB // 2
        for src_h, dst in ((nl_h, nl_v), (nr_h, nr_v)):
            for half in range(2):
                pltpu.sync_copy(
                    src_h.at[:, pl.ds(base_item + half * HALF_ITEMS, HALF_ITEMS)],
                    nraw)

                def nbody(jj, carry, dst=dst, half=half):
                    j = half * HALF_SUBS + jj
                    for n in range(NEG):
                        for h in range(SUB // LG):
                            q = n * SUB + h * LG
                            dst[j * NEG_PARTS + q // PART,
                                pl.ds(q % PART, LG)] = (
                                nraw[n, pl.ds(jj * SUB + h * LG, LG)])
                    return carry

                lax.fori_loop(0, HALF_SUBS, nbody, 0)

        lane = lax.iota(jnp.int32, 16)
        zeros = jnp.zeros((16,), jnp.float32)
        for sco in (scoA, scoB):
            for c in range(NCOLS, OUT_COLS):
                sco[c, pl.ds(0, LG)] = zeros
                sco[c, pl.ds(LG, LG)] = zeros

        def fire(j, u_h, v_h, p_v, n_v, eu, ev, en, sem):
            pltpu.async_copy(u_h.at[pu_v.at[j]], eu, sem)
            pltpu.async_copy(v_h.at[p_v.at[j]], ev, sem)
            for p in range(NEG_PARTS):
                pltpu.async_copy(v_h.at[n_v.at[j * NEG_PARTS + p]],
                                 en.at[pl.ds(p * PART, PART)], sem)

        def drain(u_h, v_h, eu, ev, en, sem):
            pltpu.make_async_copy(u_h.at[pl.ds(0, SUB)], eu, sem).wait()
            pltpu.make_async_copy(v_h.at[pl.ds(0, SUB)], ev, sem).wait()
            for p in range(NEG_PARTS):
                pltpu.make_async_copy(v_h.at[pl.ds(0, PART)],
                                      en.at[pl.ds(p * PART, PART)], sem).wait()

        def fire_l(j):
            fire(j, u_l_h, v_l_h, pvl_v, nl_v, eul, evl, enl, semL)

        def fire_r(j):
            fire(j, u_r_h, v_r_h, pvr_v, nr_v, eur, evr, enr, semR)

        def compute_side(sco, eu, ev, en, pcol, ncol0):
            for h in range(SUB // LG):
                hsl = pl.ds(h * LG, LG)
                eu_h = eu.at[pl.ds(h * LG, LG)]
                ev_h = ev.at[pl.ds(h * LG, LG)]
                groups = ((True, range(0, NEG // 2)),
                          (False, range(NEG // 2, NEG)))

                def cbody(ch, carry):
                    base = ch * DCHUNK
                    for has_pos, nrange in groups:
                        accs = {}
                        if has_pos:
                            accs["p"] = sco[pcol, hsl]
                        for n in nrange:
                            accs[n] = sco[ncol0 + n, hsl]
                        for di in range(DCHUNK):
                            dvec = jnp.full((16,), base + di, jnp.int32)
                            u = plsc.load_gather(eu_h, [lane, dvec])
                            if has_pos:
                                v = plsc.load_gather(ev_h, [lane, dvec])
                                accs["p"] = accs["p"] + u * v
                            for n in nrange:
                                nn = plsc.load_gather(
                                    en.at[pl.ds(n * SUB + h * LG, LG)],
                                    [lane, dvec])
                                accs[n] = accs[n] + u * nn
                        if has_pos:
                            sco[pcol, hsl] = accs["p"]
                        for n in nrange:
                            sco[ncol0 + n, hsl] = accs[n]
                    return carry

                lax.fori_loop(0, EMB_DIM // DCHUNK, cbody, 0)
                sco[pcol, hsl] = -sco[pcol, hsl]

        def zero_live(sco):
            for c in range(NCOLS):
                sco[c, pl.ds(0, LG)] = zeros
                sco[c, pl.ds(LG, LG)] = zeros

        def sub_chunk(j, sco, semO, t):
            fire_r(j)
            drain(u_l_h, v_l_h, eul, evl, enl, semL)

            @pl.when(t > 0)
            def _():
                pltpu.make_async_copy(sco, out_h.at[0], semO).wait()
            zero_live(sco)
            compute_side(sco, eul, evl, enl, 0, 2)

            @pl.when(j + 1 < NSUB)
            def _():
                fire_l(j + 1)
            drain(u_r_h, v_r_h, eur, evr, enr, semR)
            compute_side(sco, eur, evr, enr, 1, 2 + NEG)
            pltpu.async_copy(sco, out_h.at[wid * NSUB + j], semO)

        fire_l(0)

        def body(t, carry):
            sub_chunk(2 * t, scoA, semOA, t)
            sub_chunk(2 * t + 1, scoB, semOB, t)
            return carry

        lax.fori_loop(0, NSUB // 2, body, 0)
        pltpu.make_async_copy(scoA, out_h.at[0], semOA).wait()
        pltpu.make_async_copy(scoB, out_h.at[0], semOB).wait()

    return k(u_l, u_r, v_l, v_r, pu2, pvl2, pvr2, nl2, nr2)


def _tc_reduce(scores):
    def red(x_ref, o_ref):
        x = x_ref[...]
        s = jnp.clip(x, -10.0, 10.0)
        v = jnp.maximum(s, 0.0) + jnp.log(1.0 + jnp.exp(-jnp.abs(s)))
        col = lax.broadcasted_iota(jnp.int32, x.shape, 1)
        v = jnp.where(col < NCOLS * SUB, v, 0.0)
        o_ref[0, 0] = jnp.sum(v) * (1.0 / BATCH)

    out = pl.pallas_call(
        red,
        out_shape=jax.ShapeDtypeStruct((1, 1), jnp.float32),
        out_specs=pl.BlockSpec(memory_space=pltpu.SMEM),
    )(scores)
    return out[0, 0]


def kernel(pos_u, pos_v_l, pos_v_r, neg_v_l, neg_v_r,
           u_l_weight, u_r_weight, v_l_weight, v_r_weight):
    pu = pos_u.astype(jnp.int32)
    pvl = pos_v_l.astype(jnp.int32)
    pvr = pos_v_r.astype(jnp.int32)
    nl = neg_v_l.astype(jnp.int32).T
    nr = neg_v_r.astype(jnp.int32).T
    scores = _sc_scores(u_l_weight, u_r_weight, v_l_weight, v_r_weight,
                        pu, pvl, pvr, nl, nr)
    return _tc_reduce(scores.reshape(BATCH // SUB, OUT_COLS * SUB))


# speedup vs baseline: 1.1611x; 1.1611x over previous
import functools

import jax
import jax.numpy as jnp
from jax import lax
from jax.experimental import pallas as pl
from jax.experimental.pallas import tpu as pltpu
from jax.experimental.pallas import tpu_sc as plsc

EMB_DIM = 64
BATCH = 16384
NEG = 20
NTILES = 32
ITEMS_PER_TILE = BATCH // NTILES
SUB = 32
NSUB = ITEMS_PER_TILE // SUB
NEG_ROWS = SUB * NEG
PART = 128
NEG_PARTS = NEG_ROWS // PART
OUT_COLS = 24
NCOLS = 1 + NEG
LG = 16
DCHUNK = 16
HALF_ITEMS = ITEMS_PER_TILE // 2
HALF_SUBS = NSUB // 2


def _sc_scores_side(u_w, v_w, pu1, pv1, negT):
    mesh = plsc.VectorSubcoreMesh(core_axis_name="c", subcore_axis_name="s")

    @functools.partial(
        pl.kernel,
        out_type=jax.ShapeDtypeStruct((BATCH // SUB, OUT_COLS, SUB), jnp.float32),
        mesh=mesh,
        compiler_params=pltpu.CompilerParams(
            needs_layout_passes=False, use_tc_tiling_on_sc=False),
        scratch_types=[
            pltpu.VMEM((NSUB, SUB), jnp.int32),
            pltpu.VMEM((NSUB, SUB), jnp.int32),
            pltpu.VMEM((NSUB * NEG_PARTS, PART), jnp.int32),
            pltpu.VMEM((ITEMS_PER_TILE,), jnp.int32),
            pltpu.VMEM((NEG, HALF_ITEMS), jnp.int32),
            pltpu.VMEM((SUB, EMB_DIM), jnp.float32),
            pltpu.VMEM((SUB, EMB_DIM), jnp.float32),
            pltpu.VMEM((NEG_ROWS, EMB_DIM), jnp.float32),
            pltpu.VMEM((SUB, EMB_DIM), jnp.float32),
            pltpu.VMEM((SUB, EMB_DIM), jnp.float32),
            pltpu.VMEM((NEG_ROWS, EMB_DIM), jnp.float32),
            pltpu.VMEM((OUT_COLS, SUB), jnp.float32),
            pltpu.VMEM((OUT_COLS, SUB), jnp.float32),
            pltpu.SemaphoreType.DMA,
            pltpu.SemaphoreType.DMA,
            pltpu.SemaphoreType.DMA,
            pltpu.SemaphoreType.DMA,
        ],
    )
    def k(u_h, v_h, pu_h, pv_h, nn_h, out_h,
          pu_v, pv_v, nn_v, praw, nraw,
          euA, evA, enA, euB, evB, enB,
          scoA, scoB, semA, semB, semOA, semOB):
        wid = lax.axis_index("s") * 2 + lax.axis_index("c")
        base_item = wid * ITEMS_PER_TILE

        for src_h, dst in ((pu_h, pu_v), (pv_h, pv_v)):
            pltpu.sync_copy(src_h.at[pl.ds(base_item, ITEMS_PER_TILE)], praw)
            for j in range(NSUB):
                for h in range(SUB // LG):
                    dst[j, pl.ds(h * LG, LG)] = praw[pl.ds(j * SUB + h * LG, LG)]
        for half in range(2):
            pltpu.sync_copy(
                nn_h.at[:, pl.ds(base_item + half * HALF_ITEMS, HALF_ITEMS)],
                nraw)

            def nbody(jj, carry, half=half):
                j = half * HALF_SUBS + jj
                for n in range(NEG):
                    for h in range(SUB // LG):
                        q = n * SUB + h * LG
                        nn_v[j * NEG_PARTS + q // PART, pl.ds(q % PART, LG)] = (
                            nraw[n, pl.ds(jj * SUB + h * LG, LG)])
                return carry

            lax.fori_loop(0, HALF_SUBS, nbody, 0)

        lane = lax.iota(jnp.int32, 16)
        zeros = jnp.zeros((16,), jnp.float32)
        for sco in (scoA, scoB):
            for c in range(NCOLS, OUT_COLS):
                sco[c, pl.ds(0, LG)] = zeros
                sco[c, pl.ds(LG, LG)] = zeros

        def fire(j, eu, ev, en, sem):
            pltpu.async_copy(u_h.at[pu_v.at[j]], eu, sem)
            pltpu.async_copy(v_h.at[pv_v.at[j]], ev, sem)
            for p in range(NEG_PARTS):
                pltpu.async_copy(v_h.at[nn_v.at[j * NEG_PARTS + p]],
                                 en.at[pl.ds(p * PART, PART)], sem)

        def drain(eu, ev, en, sem):
            pltpu.make_async_copy(u_h.at[pl.ds(0, SUB)], eu, sem).wait()
            pltpu.make_async_copy(v_h.at[pl.ds(0, SUB)], ev, sem).wait()
            for p in range(NEG_PARTS):
                pltpu.make_async_copy(v_h.at[pl.ds(0, PART)],
                                      en.at[pl.ds(p * PART, PART)], sem).wait()

        def compute(j, eu, ev, en, sco, semO, t):
            @pl.when(t > 0)
            def _():
                pltpu.make_async_copy(sco, out_h.at[0], semO).wait()
            for c in range(NCOLS):
                sco[c, pl.ds(0, LG)] = zeros
                sco[c, pl.ds(LG, LG)] = zeros

            for h in range(SUB // LG):
                hsl = pl.ds(h * LG, LG)
                eu_h = eu.at[pl.ds(h * LG, LG)]
                ev_h = ev.at[pl.ds(h * LG, LG)]
                groups = ((True, range(0, NEG // 2)),
                          (False, range(NEG // 2, NEG)))

                def cbody(ch, carry):
                    base = ch * DCHUNK
                    for has_pos, nrange in groups:
                        accs = {}
                        if has_pos:
                            accs["p"] = sco[0, hsl]
                        for n in nrange:
                            accs[n] = sco[1 + n, hsl]
                        for di in range(DCHUNK):
                            dvec = jnp.full((16,), base + di, jnp.int32)
                            u = plsc.load_gather(eu_h, [lane, dvec])
                            if has_pos:
                                v = plsc.load_gather(ev_h, [lane, dvec])
                                accs["p"] = accs["p"] + u * v
                            for n in nrange:
                                nn = plsc.load_gather(
                                    en.at[pl.ds(n * SUB + h * LG, LG)],
                                    [lane, dvec])
                                accs[n] = accs[n] + u * nn
                        if has_pos:
                            sco[0, hsl] = accs["p"]
                        for n in nrange:
                            sco[1 + n, hsl] = accs[n]
                    return carry

                lax.fori_loop(0, EMB_DIM // DCHUNK, cbody, 0)
                sco[0, hsl] = -sco[0, hsl]
            pltpu.async_copy(sco, out_h.at[wid * NSUB + j], semO)

        fire(0, euA, evA, enA, semA)

        def body(t, carry):
            j0 = 2 * t
            fire(j0 + 1, euB, evB, enB, semB)
            drain(euA, evA, enA, semA)
            compute(j0, euA, evA, enA, scoA, semOA, t)

            @pl.when(j0 + 2 < NSUB)
            def _():
                fire(j0 + 2, euA, evA, enA, semA)
            drain(euB, evB, enB, semB)
            compute(j0 + 1, euB, evB, enB, scoB, semOB, t)
            return carry

        lax.fori_loop(0, NSUB // 2, body, 0)
        pltpu.make_async_copy(scoA, out_h.at[0], semOA).wait()
        pltpu.make_async_copy(scoB, out_h.at[0], semOB).wait()

    return k(u_w, v_w, pu1, pv1, negT)


def _tc_reduce(scores_l, scores_r):
    def red(x_ref, y_ref, o_ref):
        total = jnp.float32(0.0)
        for ref in (x_ref, y_ref):
            x = ref[...]
            s = jnp.clip(x, -10.0, 10.0)
            v = jnp.maximum(s, 0.0) + jnp.log(1.0 + jnp.exp(-jnp.abs(s)))
            col = lax.broadcasted_iota(jnp.int32, x.shape, 1)
            v = jnp.where(col < NCOLS * SUB, v, 0.0)
            total = total + jnp.sum(v)
        o_ref[0, 0] = total * (1.0 / BATCH)

    out = pl.pallas_call(
        red,
        out_shape=jax.ShapeDtypeStruct((1, 1), jnp.float32),
        out_specs=pl.BlockSpec(memory_space=pltpu.SMEM),
    )(scores_l, scores_r)
    return out[0, 0]


def kernel(pos_u, pos_v_l, pos_v_r, neg_v_l, neg_v_r,
           u_l_weight, u_r_weight, v_l_weight, v_r_weight):
    pu = pos_u.astype(jnp.int32)
    pvl = pos_v_l.astype(jnp.int32)
    pvr = pos_v_r.astype(jnp.int32)
    nl = neg_v_l.astype(jnp.int32).T
    nr = neg_v_r.astype(jnp.int32).T
    scores_l = _sc_scores_side(u_l_weight, v_l_weight, pu, pvl, nl)
    scores_r = _sc_scores_side(u_r_weight, v_r_weight, pu, pvr, nr)
    return _tc_reduce(scores_l.reshape(BATCH // SUB, OUT_COLS * SUB),
                      scores_r.reshape(BATCH // SUB, OUT_COLS * SUB))
